# baseline (speedup 1.0000x reference)
import jax
import jax.numpy as jnp
from jax.experimental import pallas as pl
from jax.experimental.pallas import tpu as pltpu


def _tc_copy(S, N, D, dtype):
    def body(tab_ref, out_ref, *sems):
        for n in range(N):
            pltpu.make_async_copy(tab_ref, out_ref.at[:, n, :], sems[n]).start()
        for n in range(N):
            pltpu.make_async_copy(tab_ref, out_ref.at[:, n, :], sems[n]).wait()

    return pl.pallas_call(
        body,
        out_shape=jax.ShapeDtypeStruct((S, N, D), dtype),
        in_specs=[pl.BlockSpec(memory_space=pl.ANY)],
        out_specs=pl.BlockSpec(memory_space=pl.ANY),
        scratch_shapes=[pltpu.SemaphoreType.DMA] * N,
    )


def kernel(x, pos_embedding):
    S, N = x.shape
    _, D = pos_embedding.shape
    return _tc_copy(S, N, D, pos_embedding.dtype)(pos_embedding)


# speedup vs baseline: 66.4087x; 66.4087x over previous
import jax
import jax.numpy as jnp
from jax.experimental import pallas as pl
from jax.experimental.pallas import tpu as pltpu


def _tc_copy(S, N, D, dtype, bs=256):
    def body(tab_ref, out_ref):
        rows = tab_ref[...]
        out_ref[...] = jnp.broadcast_to(rows[:, None, :], (bs, N, D))

    return pl.pallas_call(
        body,
        grid=(S // bs,),
        in_specs=[pl.BlockSpec((bs, D), lambda i: (i, 0))],
        out_specs=pl.BlockSpec((bs, N, D), lambda i: (i, 0, 0)),
        out_shape=jax.ShapeDtypeStruct((S, N, D), dtype),
    )


def kernel(x, pos_embedding):
    S, N = x.shape
    _, D = pos_embedding.shape
    return _tc_copy(S, N, D, pos_embedding.dtype)(pos_embedding)
